# baseline (speedup 1.0000x reference)
import jax
import jax.numpy as jnp
from jax.experimental import pallas as pl

_BF = jnp.bfloat16


def _dot(a, b):
    return jax.lax.dot_general(
        a, b, (((1,), (0,)), ((), ())), preferred_element_type=jnp.float32
    )


def _stage_a_kernel(x_ref, gw_ref, v1_ref, v3_ref, rl_ref, xb_ref, t1_ref, t3_ref):
    x = x_ref[...]
    rl_ref[...] = _dot(x, gw_ref[...])
    xb = x.astype(_BF)
    xb_ref[...] = xb
    t1_ref[...] = _dot(xb, v1_ref[...]).astype(_BF)
    t3_ref[...] = _dot(xb, v3_ref[...]).astype(_BF)


def _gate_up_kernel(xb_ref, w1_ref, w3_ref, u1_ref, u3_ref, t1_ref, t3_ref, h_ref):
    xb = xb_ref[...]
    gate = _dot(xb, w1_ref[...]) + _dot(t1_ref[...], u1_ref[...])
    up = _dot(xb, w3_ref[...]) + _dot(t3_ref[...], u3_ref[...])
    h_ref[...] = (jax.nn.silu(gate) * up).astype(_BF)


def _down_kernel(h_ref, v2_ref, w2_ref, u2_ref, o_ref):
    h = h_ref[...]
    t2 = _dot(h, v2_ref[...]).astype(_BF)
    o_ref[...] = _dot(h, w2_ref[...]) + _dot(t2, u2_ref[...])


def kernel(hidden_states, gate_w, w1, w2, w3, u1, v1, u2, v2, u3, v3):
    b, s, d = hidden_states.shape
    T = b * s
    H = d
    F = w1.shape[0]
    R = u1.shape[1]
    E = gate_w.shape[0]
    x = hidden_states.reshape(T, H)

    gwt = gate_w.T
    w1t, w3t = w1.T.astype(_BF), w3.T.astype(_BF)
    w2t = w2.T.astype(_BF)
    u1t, u3t = u1.T.astype(_BF), u3.T.astype(_BF)
    u2t = u2.T.astype(_BF)
    v1t, v3t = v1.T.astype(_BF), v3.T.astype(_BF)
    v2t = v2.T.astype(_BF)

    tM = min(512, T)
    nM = T // tM

    rl, xb, t1, t3 = pl.pallas_call(
        _stage_a_kernel,
        grid=(nM,),
        in_specs=[
            pl.BlockSpec((tM, H), lambda m: (m, 0)),
            pl.BlockSpec((H, E), lambda m: (0, 0)),
            pl.BlockSpec((H, R), lambda m: (0, 0)),
            pl.BlockSpec((H, R), lambda m: (0, 0)),
        ],
        out_specs=[
            pl.BlockSpec((tM, E), lambda m: (m, 0)),
            pl.BlockSpec((tM, H), lambda m: (m, 0)),
            pl.BlockSpec((tM, R), lambda m: (m, 0)),
            pl.BlockSpec((tM, R), lambda m: (m, 0)),
        ],
        out_shape=[
            jax.ShapeDtypeStruct((T, E), jnp.float32),
            jax.ShapeDtypeStruct((T, H), _BF),
            jax.ShapeDtypeStruct((T, R), _BF),
            jax.ShapeDtypeStruct((T, R), _BF),
        ],
    )(x, gwt, v1t, v3t)

    tF = min(1024, F)
    nF = F // tF
    h = pl.pallas_call(
        _gate_up_kernel,
        grid=(nF, nM),
        in_specs=[
            pl.BlockSpec((tM, H), lambda f, m: (m, 0)),
            pl.BlockSpec((H, tF), lambda f, m: (0, f)),
            pl.BlockSpec((H, tF), lambda f, m: (0, f)),
            pl.BlockSpec((R, tF), lambda f, m: (0, f)),
            pl.BlockSpec((R, tF), lambda f, m: (0, f)),
            pl.BlockSpec((tM, R), lambda f, m: (m, 0)),
            pl.BlockSpec((tM, R), lambda f, m: (m, 0)),
        ],
        out_specs=pl.BlockSpec((tM, tF), lambda f, m: (m, f)),
        out_shape=jax.ShapeDtypeStruct((T, F), _BF),
    )(xb, w1t, w3t, u1t, u3t, t1, t3)

    tH = min(1024, H)
    nH = H // tH
    out = pl.pallas_call(
        _down_kernel,
        grid=(nH, nM),
        in_specs=[
            pl.BlockSpec((tM, F), lambda hh, m: (m, 0)),
            pl.BlockSpec((F, R), lambda hh, m: (0, 0)),
            pl.BlockSpec((F, tH), lambda hh, m: (0, hh)),
            pl.BlockSpec((R, tH), lambda hh, m: (0, hh)),
        ],
        out_specs=pl.BlockSpec((tM, tH), lambda hh, m: (m, hh)),
        out_shape=jax.ShapeDtypeStruct((T, H), jnp.float32),
    )(h, v2t, w2t, u2t)

    return out.reshape(b, s, d), rl


# speedup vs baseline: 1.3724x; 1.3724x over previous
import jax
import jax.numpy as jnp
from jax.experimental import pallas as pl
from jax.experimental.pallas import tpu as pltpu

_BF = jnp.bfloat16


def _dot_t(a, b):
    return jax.lax.dot_general(
        a, b, (((1,), (1,)), ((), ())), preferred_element_type=jnp.float32
    )


def _dot(a, b):
    return jax.lax.dot_general(
        a, b, (((1,), (0,)), ((), ())), preferred_element_type=jnp.float32
    )


def _stage_a_kernel(x_ref, gw_ref, rl_ref, xb_ref):
    x = x_ref[...]
    rl_ref[...] = _dot_t(x, gw_ref[...])
    xb_ref[...] = x.astype(_BF)


def _gate_up_kernel(xb_ref, w1_ref, w3_ref, u1_ref, u3_ref, v1_ref, v3_ref,
                    h_ref, m1_ref, m3_ref):
    @pl.when(pl.program_id(1) == 0)
    def _merge():
        v1b = v1_ref[...].astype(_BF)
        v3b = v3_ref[...].astype(_BF)
        m1_ref[...] = (
            w1_ref[...] + _dot(u1_ref[...].astype(_BF), v1b)
        ).astype(_BF)
        m3_ref[...] = (
            w3_ref[...] + _dot(u3_ref[...].astype(_BF), v3b)
        ).astype(_BF)

    xb = xb_ref[...]
    gate = _dot_t(xb, m1_ref[...])
    up = _dot_t(xb, m3_ref[...])
    h_ref[...] = (jax.nn.silu(gate) * up).astype(_BF)


def _down_kernel(h_ref, w2_ref, u2_ref, v2_ref, o_ref, m2_ref):
    @pl.when(pl.program_id(1) == 0)
    def _merge():
        m2_ref[...] = (
            w2_ref[...] + _dot(u2_ref[...].astype(_BF), v2_ref[...].astype(_BF))
        ).astype(_BF)

    o_ref[...] = _dot_t(h_ref[...], m2_ref[...])


def kernel(hidden_states, gate_w, w1, w2, w3, u1, v1, u2, v2, u3, v3):
    b, s, d = hidden_states.shape
    T = b * s
    H = d
    F = w1.shape[0]
    R = u1.shape[1]
    E = gate_w.shape[0]
    x = hidden_states.reshape(T, H)

    tMa = min(1024, T)
    nMa = T // tMa

    rl, xb = pl.pallas_call(
        _stage_a_kernel,
        grid=(nMa,),
        in_specs=[
            pl.BlockSpec((tMa, H), lambda m: (m, 0)),
            pl.BlockSpec((E, H), lambda m: (0, 0)),
        ],
        out_specs=[
            pl.BlockSpec((tMa, E), lambda m: (m, 0)),
            pl.BlockSpec((tMa, H), lambda m: (m, 0)),
        ],
        out_shape=[
            jax.ShapeDtypeStruct((T, E), jnp.float32),
            jax.ShapeDtypeStruct((T, H), _BF),
        ],
    )(x, gate_w)

    tM = min(1024, T)
    nM = T // tM
    tF = min(512, F)
    nF = F // tF
    h = pl.pallas_call(
        _gate_up_kernel,
        grid=(nF, nM),
        in_specs=[
            pl.BlockSpec((tM, H), lambda f, m: (m, 0)),
            pl.BlockSpec((tF, H), lambda f, m: (f, 0)),
            pl.BlockSpec((tF, H), lambda f, m: (f, 0)),
            pl.BlockSpec((tF, R), lambda f, m: (f, 0)),
            pl.BlockSpec((tF, R), lambda f, m: (f, 0)),
            pl.BlockSpec((R, H), lambda f, m: (0, 0)),
            pl.BlockSpec((R, H), lambda f, m: (0, 0)),
        ],
        out_specs=pl.BlockSpec((tM, tF), lambda f, m: (m, f)),
        out_shape=jax.ShapeDtypeStruct((T, F), _BF),
        scratch_shapes=[
            pltpu.VMEM((tF, H), _BF),
            pltpu.VMEM((tF, H), _BF),
        ],
    )(xb, w1, w3, u1, u3, v1, v3)

    tH = min(512, H)
    nH = H // tH
    out = pl.pallas_call(
        _down_kernel,
        grid=(nH, nM),
        in_specs=[
            pl.BlockSpec((tM, F), lambda hh, m: (m, 0)),
            pl.BlockSpec((tH, F), lambda hh, m: (hh, 0)),
            pl.BlockSpec((tH, R), lambda hh, m: (hh, 0)),
            pl.BlockSpec((R, F), lambda hh, m: (0, 0)),
        ],
        out_specs=pl.BlockSpec((tM, tH), lambda hh, m: (m, hh)),
        out_shape=jax.ShapeDtypeStruct((T, H), jnp.float32),
        scratch_shapes=[
            pltpu.VMEM((tH, F), _BF),
        ],
    )(h, w2, u2, v2)

    return out.reshape(b, s, d), rl
